# Initial kernel scaffold; baseline (speedup 1.0000x reference)
import jax
import jax.numpy as jnp
from jax.experimental import pallas as pl


def kernel(x, edge_type, means, stds, mul_weight, bias_weight):
    raise NotImplementedError("write your pallas kernel here")


# speedup vs baseline: 20.4565x; 20.4565x over previous
import dataclasses
import functools
import math

import jax
import jax.numpy as jnp
from jax import lax
from jax.experimental import pallas as pl
from jax.experimental.pallas import tpu as pltpu
from jax.experimental.pallas import tpu_sc as plsc

_LANES = 16
_NW = 32


def _sc_gather_affine(xf, ef, mul_t, bias_t):
    n = xf.shape[0]
    per_w = n // _NW
    e = mul_t.shape[0]
    mesh = plsc.VectorSubcoreMesh(core_axis_name="c", subcore_axis_name="s")
    cp = pltpu.CompilerParams()
    if "needs_layout_passes" in pltpu.CompilerParams.__dataclass_fields__:
        cp = dataclasses.replace(cp, needs_layout_passes=False)

    @functools.partial(
        pl.kernel,
        compiler_params=cp,
        out_type=jax.ShapeDtypeStruct((n,), jnp.float32),
        mesh=mesh,
        scratch_types=[
            pltpu.VMEM((per_w,), jnp.int32),
            pltpu.VMEM((per_w,), jnp.float32),
            pltpu.VMEM((per_w,), jnp.float32),
            pltpu.VMEM((e,), jnp.float32),
            pltpu.VMEM((e,), jnp.float32),
        ],
    )
    def k(x_hbm, e_hbm, mul_hbm, bias_hbm, out_hbm, idx_v, x_v, out_v,
          mul_v, bias_v):
        wid = lax.axis_index("s") * 2 + lax.axis_index("c")
        base = wid * per_w
        pltpu.sync_copy(mul_hbm, mul_v)
        pltpu.sync_copy(bias_hbm, bias_v)
        pltpu.sync_copy(e_hbm.at[pl.ds(base, per_w)], idx_v)
        pltpu.sync_copy(x_hbm.at[pl.ds(base, per_w)], x_v)

        @pl.loop(0, per_w, step=_LANES)
        def _(j):
            iv = idx_v[pl.ds(j, _LANES)]
            mv = plsc.load_gather(mul_v, [iv])
            bv = plsc.load_gather(bias_v, [iv])
            out_v[pl.ds(j, _LANES)] = mv * x_v[pl.ds(j, _LANES)] + bv

        pltpu.sync_copy(out_v, out_hbm.at[pl.ds(base, per_w)])

    return k(xf, ef, mul_t, bias_t)


def _tc_body(x_ref, c_ref, o_ref, *, m_rows, k):
    xv = x_ref[...]
    mean = c_ref[0:1, :]
    q = c_ref[1:2, :]
    coef = c_ref[2:3, :]
    xb = jnp.broadcast_to(xv, (m_rows, k))
    d = xb - mean
    o_ref[...] = coef * jnp.exp2(d * d * q)


def _tc_gaussian(xx, consts, m_rows):
    n = xx.shape[0]
    k = consts.shape[1]
    return pl.pallas_call(
        functools.partial(_tc_body, m_rows=m_rows, k=k),
        grid=(n // m_rows,),
        in_specs=[
            pl.BlockSpec((m_rows, 1), lambda i: (i, 0)),
            pl.BlockSpec((8, k), lambda i: (0, 0)),
        ],
        out_specs=pl.BlockSpec((m_rows, k), lambda i: (i, 0)),
        out_shape=jax.ShapeDtypeStruct((n, k), jnp.float32),
    )(xx.reshape(n, 1), consts)


def kernel(x, edge_type, means, stds, mul_weight, bias_weight):
    b, n, _ = x.shape
    k = means.shape[0]
    bnn = b * n * n

    xx = _sc_gather_affine(
        x.reshape(bnn),
        edge_type.reshape(bnn),
        mul_weight.reshape(-1),
        bias_weight.reshape(-1),
    )

    a = 1.0 / math.sqrt(2.0 * math.pi)
    log2e = math.log2(math.e)
    inv = 1.0 / (stds + 1e-6)
    consts = jnp.zeros((8, k), jnp.float32)
    consts = consts.at[0].set(means)
    consts = consts.at[1].set(-0.5 * log2e * inv * inv)
    consts = consts.at[2].set(a * inv)

    out = _tc_gaussian(xx, consts, m_rows=2048)
    return out.reshape(b, n, n, k)


# speedup vs baseline: 24.1564x; 1.1809x over previous
import dataclasses
import functools
import math

import jax
import jax.numpy as jnp
from jax import lax
from jax.experimental import pallas as pl
from jax.experimental.pallas import tpu as pltpu
from jax.experimental.pallas import tpu_sc as plsc

_LANES = 16
_NW = 32


def _sc_gather_affine(xf, ef, mul_t, bias_t):
    n = xf.shape[0]
    per_w = n // _NW
    e = mul_t.shape[0]
    mesh = plsc.VectorSubcoreMesh(core_axis_name="c", subcore_axis_name="s")
    cp = pltpu.CompilerParams()
    if "needs_layout_passes" in pltpu.CompilerParams.__dataclass_fields__:
        cp = dataclasses.replace(cp, needs_layout_passes=False)

    @functools.partial(
        pl.kernel,
        compiler_params=cp,
        out_type=jax.ShapeDtypeStruct((n,), jnp.float32),
        mesh=mesh,
        scratch_types=[
            pltpu.VMEM((per_w,), jnp.int32),
            pltpu.VMEM((per_w,), jnp.float32),
            pltpu.VMEM((per_w,), jnp.float32),
            pltpu.VMEM((e,), jnp.float32),
            pltpu.VMEM((e,), jnp.float32),
        ],
    )
    def k(x_hbm, e_hbm, mul_hbm, bias_hbm, out_hbm, idx_v, x_v, out_v,
          mul_v, bias_v):
        wid = lax.axis_index("s") * 2 + lax.axis_index("c")
        base = wid * per_w
        pltpu.sync_copy(mul_hbm, mul_v)
        pltpu.sync_copy(bias_hbm, bias_v)
        pltpu.sync_copy(e_hbm.at[pl.ds(base, per_w)], idx_v)
        pltpu.sync_copy(x_hbm.at[pl.ds(base, per_w)], x_v)

        @pl.loop(0, per_w, step=_LANES)
        def _(j):
            iv = idx_v[pl.ds(j, _LANES)]
            mv = plsc.load_gather(mul_v, [iv])
            bv = plsc.load_gather(bias_v, [iv])
            out_v[pl.ds(j, _LANES)] = mv * x_v[pl.ds(j, _LANES)] + bv

        pltpu.sync_copy(out_v, out_hbm.at[pl.ds(base, per_w)])

    return k(xf, ef, mul_t, bias_t)


def _tc_body(x_ref, c_ref, o_ref, *, bg, k):
    xv = x_ref[...]
    mean = c_ref[0:1, :]
    q = c_ref[1:2, :]
    coef = c_ref[2:3, :]
    xt = xv.T
    for s in range(bg):
        xcol = xt[:, s:s + 1]
        xb = jnp.broadcast_to(xcol, (128, k))
        d = xb - mean
        o_ref[s] = coef * jnp.exp2(d * d * q)


def _tc_gaussian(xx, consts, bg):
    n = xx.shape[0]
    k = consts.shape[1]
    g = n // 128
    return pl.pallas_call(
        functools.partial(_tc_body, bg=bg, k=k),
        grid=(g // bg,),
        in_specs=[
            pl.BlockSpec((bg, 128), lambda i: (i, 0)),
            pl.BlockSpec((8, k), lambda i: (0, 0)),
        ],
        out_specs=pl.BlockSpec((bg, 128, k), lambda i: (i, 0, 0)),
        out_shape=jax.ShapeDtypeStruct((g, 128, k), jnp.float32),
    )(xx.reshape(g, 128), consts)


def kernel(x, edge_type, means, stds, mul_weight, bias_weight):
    b, n, _ = x.shape
    k = means.shape[0]
    bnn = b * n * n

    xx = _sc_gather_affine(
        x.reshape(bnn),
        edge_type.reshape(bnn),
        mul_weight.reshape(-1),
        bias_weight.reshape(-1),
    )

    a = 1.0 / math.sqrt(2.0 * math.pi)
    log2e = math.log2(math.e)
    inv = 1.0 / (stds + 1e-6)
    consts = jnp.zeros((8, k), jnp.float32)
    consts = consts.at[0].set(means)
    consts = consts.at[1].set(-0.5 * log2e * inv * inv)
    consts = consts.at[2].set(a * inv)

    out = _tc_gaussian(xx, consts, bg=8)
    return out.reshape(b, n, n, k)


# speedup vs baseline: 36.4488x; 1.5089x over previous
import dataclasses
import functools
import math

import jax
import jax.numpy as jnp
from jax import lax
from jax.experimental import pallas as pl
from jax.experimental.pallas import tpu as pltpu
from jax.experimental.pallas import tpu_sc as plsc

_LANES = 16
_NW = 32


def _sc_gather_affine(xf, ef, mul_t, bias_t):
    n = xf.shape[0]
    per_w = n // _NW
    e = mul_t.shape[0]
    mesh = plsc.VectorSubcoreMesh(core_axis_name="c", subcore_axis_name="s")
    cp = pltpu.CompilerParams()
    if "needs_layout_passes" in pltpu.CompilerParams.__dataclass_fields__:
        cp = dataclasses.replace(cp, needs_layout_passes=False)

    @functools.partial(
        pl.kernel,
        compiler_params=cp,
        out_type=jax.ShapeDtypeStruct((n,), jnp.float32),
        mesh=mesh,
        scratch_types=[
            pltpu.VMEM((per_w,), jnp.int32),
            pltpu.VMEM((per_w,), jnp.float32),
            pltpu.VMEM((per_w,), jnp.float32),
            pltpu.VMEM((e,), jnp.float32),
            pltpu.VMEM((e,), jnp.float32),
        ],
    )
    def k(x_hbm, e_hbm, mul_hbm, bias_hbm, out_hbm, idx_v, x_v, out_v,
          mul_v, bias_v):
        wid = lax.axis_index("s") * 2 + lax.axis_index("c")
        base = wid * per_w
        pltpu.sync_copy(mul_hbm, mul_v)
        pltpu.sync_copy(bias_hbm, bias_v)
        pltpu.sync_copy(e_hbm.at[pl.ds(base, per_w)], idx_v)
        pltpu.sync_copy(x_hbm.at[pl.ds(base, per_w)], x_v)

        @pl.loop(0, per_w, step=_LANES)
        def _(j):
            iv = idx_v[pl.ds(j, _LANES)]
            mv = plsc.load_gather(mul_v, [iv])
            bv = plsc.load_gather(bias_v, [iv])
            out_v[pl.ds(j, _LANES)] = mv * x_v[pl.ds(j, _LANES)] + bv

        pltpu.sync_copy(out_v, out_hbm.at[pl.ds(base, per_w)])

    return k(xf, ef, mul_t, bias_t)


def _tc_body(x_ref, c_ref, o_ref, *, bg, k):
    xv = x_ref[...]
    mean = c_ref[0:1, :]
    q = c_ref[1:2, :]
    coef = c_ref[2:3, :]
    xt = xv.T
    for s in range(bg):
        xcol = xt[:, s:s + 1]
        xb = jnp.broadcast_to(xcol, (128, k))
        d = xb - mean
        o_ref[s] = coef * jnp.exp2(d * d * q)


def _tc_gaussian(xx, consts, bg):
    n = xx.shape[0]
    k = consts.shape[1]
    g = n // 128
    return pl.pallas_call(
        functools.partial(_tc_body, bg=bg, k=k),
        grid=(g // bg,),
        in_specs=[
            pl.BlockSpec((bg, 128), lambda i: (i, 0)),
            pl.BlockSpec((8, k), lambda i: (0, 0)),
        ],
        out_specs=pl.BlockSpec((bg, 128, k), lambda i: (i, 0, 0)),
        out_shape=jax.ShapeDtypeStruct((g, 128, k), jnp.float32),
    )(xx.reshape(g, 128), consts)


def kernel(x, edge_type, means, stds, mul_weight, bias_weight):
    b, n, _ = x.shape
    k = means.shape[0]
    bnn = b * n * n

    xx = _sc_gather_affine(
        x.reshape(bnn),
        edge_type.reshape(bnn),
        mul_weight.reshape(-1),
        bias_weight.reshape(-1),
    )

    a = 1.0 / math.sqrt(2.0 * math.pi)
    log2e = math.log2(math.e)
    inv = 1.0 / (stds + 1e-6)
    consts = jnp.zeros((8, k), jnp.float32)
    consts = consts.at[0].set(means)
    consts = consts.at[1].set(-0.5 * log2e * inv * inv)
    consts = consts.at[2].set(a * inv)

    out = _tc_gaussian(xx, consts, bg=16)
    return out.reshape(b, n, n, k)


# speedup vs baseline: 50.3104x; 1.3803x over previous
import dataclasses
import functools
import math

import jax
import jax.numpy as jnp
from jax import lax
from jax.experimental import pallas as pl
from jax.experimental.pallas import tpu as pltpu
from jax.experimental.pallas import tpu_sc as plsc

_LANES = 16
_NW = 32


def _sc_gather_affine(xf, ef, mul_t, bias_t):
    n = xf.shape[0]
    per_w = n // _NW
    e = mul_t.shape[0]
    mesh = plsc.VectorSubcoreMesh(core_axis_name="c", subcore_axis_name="s")
    cp = pltpu.CompilerParams()
    if "needs_layout_passes" in pltpu.CompilerParams.__dataclass_fields__:
        cp = dataclasses.replace(cp, needs_layout_passes=False)

    @functools.partial(
        pl.kernel,
        compiler_params=cp,
        out_type=jax.ShapeDtypeStruct((n,), jnp.float32),
        mesh=mesh,
        scratch_types=[
            pltpu.VMEM((per_w,), jnp.int32),
            pltpu.VMEM((per_w,), jnp.float32),
            pltpu.VMEM((per_w,), jnp.float32),
            pltpu.VMEM((e,), jnp.float32),
            pltpu.VMEM((e,), jnp.float32),
        ],
    )
    def k(x_hbm, e_hbm, mul_hbm, bias_hbm, out_hbm, idx_v, x_v, out_v,
          mul_v, bias_v):
        wid = lax.axis_index("s") * 2 + lax.axis_index("c")
        base = wid * per_w
        pltpu.sync_copy(mul_hbm, mul_v)
        pltpu.sync_copy(bias_hbm, bias_v)
        pltpu.sync_copy(e_hbm.at[pl.ds(base, per_w)], idx_v)
        pltpu.sync_copy(x_hbm.at[pl.ds(base, per_w)], x_v)

        @pl.loop(0, per_w, step=_LANES)
        def _(j):
            iv = idx_v[pl.ds(j, _LANES)]
            mv = plsc.load_gather(mul_v, [iv])
            bv = plsc.load_gather(bias_v, [iv])
            out_v[pl.ds(j, _LANES)] = mv * x_v[pl.ds(j, _LANES)] + bv

        pltpu.sync_copy(out_v, out_hbm.at[pl.ds(base, per_w)])

    return k(xf, ef, mul_t, bias_t)


def _tc_body(x_ref, c_ref, o_ref, *, bg, k):
    xv = x_ref[...]
    mean = c_ref[0:1, :]
    q = c_ref[1:2, :]
    coef = c_ref[2:3, :]
    xt = xv.T
    for s in range(bg):
        xcol = xt[:, s:s + 1]
        xb = jnp.broadcast_to(xcol, (128, k))
        d = xb - mean
        o_ref[s] = coef * jnp.exp2(d * d * q)


def _tc_gaussian(xx, consts, bg):
    n = xx.shape[0]
    k = consts.shape[1]
    g = n // 128
    return pl.pallas_call(
        functools.partial(_tc_body, bg=bg, k=k),
        grid=(g // bg,),
        in_specs=[
            pl.BlockSpec((bg, 128), lambda i: (i, 0)),
            pl.BlockSpec((8, k), lambda i: (0, 0)),
        ],
        out_specs=pl.BlockSpec((bg, 128, k), lambda i: (i, 0, 0)),
        out_shape=jax.ShapeDtypeStruct((g, 128, k), jnp.float32),
    )(xx.reshape(g, 128), consts)


def kernel(x, edge_type, means, stds, mul_weight, bias_weight):
    b, n, _ = x.shape
    k = means.shape[0]
    bnn = b * n * n

    xx = _sc_gather_affine(
        x.reshape(bnn),
        edge_type.reshape(bnn),
        mul_weight.reshape(-1),
        bias_weight.reshape(-1),
    )

    a = 1.0 / math.sqrt(2.0 * math.pi)
    log2e = math.log2(math.e)
    inv = 1.0 / (stds + 1e-6)
    consts = jnp.zeros((8, k), jnp.float32)
    consts = consts.at[0].set(means)
    consts = consts.at[1].set(-0.5 * log2e * inv * inv)
    consts = consts.at[2].set(a * inv)

    out = _tc_gaussian(xx, consts, bg=32)
    return out.reshape(b, n, n, k)


# speedup vs baseline: 61.0818x; 1.2141x over previous
import dataclasses
import functools
import math

import jax
import jax.numpy as jnp
from jax import lax
from jax.experimental import pallas as pl
from jax.experimental.pallas import tpu as pltpu
from jax.experimental.pallas import tpu_sc as plsc

_LANES = 16
_NW = 32


def _sc_gather_affine(xf, ef, mul_t, bias_t):
    n = xf.shape[0]
    per_w = n // _NW
    e = mul_t.shape[0]
    mesh = plsc.VectorSubcoreMesh(core_axis_name="c", subcore_axis_name="s")
    cp = pltpu.CompilerParams()
    if "needs_layout_passes" in pltpu.CompilerParams.__dataclass_fields__:
        cp = dataclasses.replace(cp, needs_layout_passes=False)

    @functools.partial(
        pl.kernel,
        compiler_params=cp,
        out_type=jax.ShapeDtypeStruct((n,), jnp.float32),
        mesh=mesh,
        scratch_types=[
            pltpu.VMEM((per_w,), jnp.int32),
            pltpu.VMEM((per_w,), jnp.float32),
            pltpu.VMEM((per_w,), jnp.float32),
            pltpu.VMEM((e,), jnp.float32),
            pltpu.VMEM((e,), jnp.float32),
        ],
    )
    def k(x_hbm, e_hbm, mul_hbm, bias_hbm, out_hbm, idx_v, x_v, out_v,
          mul_v, bias_v):
        wid = lax.axis_index("s") * 2 + lax.axis_index("c")
        base = wid * per_w
        pltpu.sync_copy(mul_hbm, mul_v)
        pltpu.sync_copy(bias_hbm, bias_v)
        pltpu.sync_copy(e_hbm.at[pl.ds(base, per_w)], idx_v)
        pltpu.sync_copy(x_hbm.at[pl.ds(base, per_w)], x_v)

        @pl.loop(0, per_w, step=_LANES)
        def _(j):
            iv = idx_v[pl.ds(j, _LANES)]
            mv = plsc.load_gather(mul_v, [iv])
            bv = plsc.load_gather(bias_v, [iv])
            out_v[pl.ds(j, _LANES)] = mv * x_v[pl.ds(j, _LANES)] + bv

        pltpu.sync_copy(out_v, out_hbm.at[pl.ds(base, per_w)])

    return k(xf, ef, mul_t, bias_t)


def _tc_body(x_ref, c_ref, o_ref, *, bg, k):
    xv = x_ref[...]
    mean = c_ref[0:1, :]
    q = c_ref[1:2, :]
    coef = c_ref[2:3, :]
    xt = xv.T
    for s in range(bg):
        xcol = xt[:, s:s + 1]
        xb = jnp.broadcast_to(xcol, (128, k))
        d = xb - mean
        o_ref[s] = coef * jnp.exp2(d * d * q)


def _tc_gaussian(xx, consts, bg):
    n = xx.shape[0]
    k = consts.shape[1]
    g = n // 128
    return pl.pallas_call(
        functools.partial(_tc_body, bg=bg, k=k),
        grid=(g // bg,),
        in_specs=[
            pl.BlockSpec((bg, 128), lambda i: (i, 0)),
            pl.BlockSpec((8, k), lambda i: (0, 0)),
        ],
        out_specs=pl.BlockSpec((bg, 128, k), lambda i: (i, 0, 0)),
        out_shape=jax.ShapeDtypeStruct((g, 128, k), jnp.float32),
    )(xx.reshape(g, 128), consts)


def kernel(x, edge_type, means, stds, mul_weight, bias_weight):
    b, n, _ = x.shape
    k = means.shape[0]
    bnn = b * n * n

    xx = _sc_gather_affine(
        x.reshape(bnn),
        edge_type.reshape(bnn),
        mul_weight.reshape(-1),
        bias_weight.reshape(-1),
    )

    a = 1.0 / math.sqrt(2.0 * math.pi)
    log2e = math.log2(math.e)
    inv = 1.0 / (stds + 1e-6)
    consts = jnp.zeros((8, k), jnp.float32)
    consts = consts.at[0].set(means)
    consts = consts.at[1].set(-0.5 * log2e * inv * inv)
    consts = consts.at[2].set(a * inv)

    out = _tc_gaussian(xx, consts, bg=64)
    return out.reshape(b, n, n, k)


# speedup vs baseline: 66.6679x; 1.0915x over previous
import dataclasses
import functools
import math

import jax
import jax.numpy as jnp
from jax import lax
from jax.experimental import pallas as pl
from jax.experimental.pallas import tpu as pltpu
from jax.experimental.pallas import tpu_sc as plsc

_LANES = 16
_NW = 32


def _sc_gather_affine(xf, ef, mul_t, bias_t):
    n = xf.shape[0]
    per_w = n // _NW
    e = mul_t.shape[0]
    mesh = plsc.VectorSubcoreMesh(core_axis_name="c", subcore_axis_name="s")
    cp = pltpu.CompilerParams()
    if "needs_layout_passes" in pltpu.CompilerParams.__dataclass_fields__:
        cp = dataclasses.replace(cp, needs_layout_passes=False)

    @functools.partial(
        pl.kernel,
        compiler_params=cp,
        out_type=jax.ShapeDtypeStruct((n,), jnp.float32),
        mesh=mesh,
        scratch_types=[
            pltpu.VMEM((per_w,), jnp.int32),
            pltpu.VMEM((per_w,), jnp.float32),
            pltpu.VMEM((per_w,), jnp.float32),
            pltpu.VMEM((e,), jnp.float32),
            pltpu.VMEM((e,), jnp.float32),
        ],
    )
    def k(x_hbm, e_hbm, mul_hbm, bias_hbm, out_hbm, idx_v, x_v, out_v,
          mul_v, bias_v):
        wid = lax.axis_index("s") * 2 + lax.axis_index("c")
        base = wid * per_w
        pltpu.sync_copy(mul_hbm, mul_v)
        pltpu.sync_copy(bias_hbm, bias_v)
        pltpu.sync_copy(e_hbm.at[pl.ds(base, per_w)], idx_v)
        pltpu.sync_copy(x_hbm.at[pl.ds(base, per_w)], x_v)

        @pl.loop(0, per_w, step=_LANES)
        def _(j):
            iv = idx_v[pl.ds(j, _LANES)]
            mv = plsc.load_gather(mul_v, [iv])
            bv = plsc.load_gather(bias_v, [iv])
            out_v[pl.ds(j, _LANES)] = mv * x_v[pl.ds(j, _LANES)] + bv

        pltpu.sync_copy(out_v, out_hbm.at[pl.ds(base, per_w)])

    return k(xf, ef, mul_t, bias_t)


def _tc_body(x_ref, c_ref, o_ref, *, bg, k):
    xv = x_ref[...]
    mean = c_ref[0:1, :]
    q = c_ref[1:2, :]
    coef = c_ref[2:3, :]
    xt = xv.T
    for s in range(bg):
        xcol = xt[:, s:s + 1]
        xb = jnp.broadcast_to(xcol, (128, k))
        d = xb - mean
        o_ref[s] = coef * jnp.exp2(d * d * q)


def _tc_gaussian(xx, consts, bg):
    n = xx.shape[0]
    k = consts.shape[1]
    g = n // 128
    return pl.pallas_call(
        functools.partial(_tc_body, bg=bg, k=k),
        grid=(g // bg,),
        in_specs=[
            pl.BlockSpec((bg, 128), lambda i: (i, 0)),
            pl.BlockSpec((8, k), lambda i: (0, 0)),
        ],
        out_specs=pl.BlockSpec((bg, 128, k), lambda i: (i, 0, 0)),
        out_shape=jax.ShapeDtypeStruct((g, 128, k), jnp.float32),
    )(xx.reshape(g, 128), consts)


def kernel(x, edge_type, means, stds, mul_weight, bias_weight):
    b, n, _ = x.shape
    k = means.shape[0]
    bnn = b * n * n

    xx = _sc_gather_affine(
        x.reshape(bnn),
        edge_type.reshape(bnn),
        mul_weight.reshape(-1),
        bias_weight.reshape(-1),
    )

    a = 1.0 / math.sqrt(2.0 * math.pi)
    log2e = math.log2(math.e)
    inv = 1.0 / (stds + 1e-6)
    consts = jnp.zeros((8, k), jnp.float32)
    consts = consts.at[0].set(means)
    consts = consts.at[1].set(-0.5 * log2e * inv * inv)
    consts = consts.at[2].set(a * inv)

    out = _tc_gaussian(xx, consts, bg=128)
    return out.reshape(b, n, n, k)


# speedup vs baseline: 68.7316x; 1.0310x over previous
import dataclasses
import functools
import math

import jax
import jax.numpy as jnp
from jax import lax
from jax.experimental import pallas as pl
from jax.experimental.pallas import tpu as pltpu
from jax.experimental.pallas import tpu_sc as plsc

_LANES = 16
_NW = 32


def _sc_gather_affine(xf, ef, mul_t, bias_t):
    n = xf.shape[0]
    per_w = n // _NW
    e = mul_t.shape[0]
    mesh = plsc.VectorSubcoreMesh(core_axis_name="c", subcore_axis_name="s")
    cp = pltpu.CompilerParams()
    if "needs_layout_passes" in pltpu.CompilerParams.__dataclass_fields__:
        cp = dataclasses.replace(cp, needs_layout_passes=False)

    @functools.partial(
        pl.kernel,
        compiler_params=cp,
        out_type=jax.ShapeDtypeStruct((n,), jnp.float32),
        mesh=mesh,
        scratch_types=[
            pltpu.VMEM((per_w,), jnp.int32),
            pltpu.VMEM((per_w,), jnp.float32),
            pltpu.VMEM((per_w,), jnp.float32),
            pltpu.VMEM((e,), jnp.float32),
            pltpu.VMEM((e,), jnp.float32),
        ],
    )
    def k(x_hbm, e_hbm, mul_hbm, bias_hbm, out_hbm, idx_v, x_v, out_v,
          mul_v, bias_v):
        wid = lax.axis_index("s") * 2 + lax.axis_index("c")
        base = wid * per_w
        pltpu.sync_copy(mul_hbm, mul_v)
        pltpu.sync_copy(bias_hbm, bias_v)
        pltpu.sync_copy(e_hbm.at[pl.ds(base, per_w)], idx_v)
        pltpu.sync_copy(x_hbm.at[pl.ds(base, per_w)], x_v)

        @pl.loop(0, per_w, step=_LANES)
        def _(j):
            iv = idx_v[pl.ds(j, _LANES)]
            mv = plsc.load_gather(mul_v, [iv])
            bv = plsc.load_gather(bias_v, [iv])
            out_v[pl.ds(j, _LANES)] = mv * x_v[pl.ds(j, _LANES)] + bv

        pltpu.sync_copy(out_v, out_hbm.at[pl.ds(base, per_w)])

    return k(xf, ef, mul_t, bias_t)


def _tc_body(x_ref, c_ref, o_ref, *, bg, k):
    xv = x_ref[...]
    mean = c_ref[0:1, :]
    q = c_ref[1:2, :]
    coef = c_ref[2:3, :]
    xt = xv.T
    for s in range(bg):
        xcol = xt[:, s:s + 1]
        xb = jnp.broadcast_to(xcol, (128, k))
        d = xb - mean
        o_ref[s] = coef * jnp.exp2(d * d * q)


def _tc_gaussian(xx, consts, bg):
    n = xx.shape[0]
    k = consts.shape[1]
    g = n // 128
    return pl.pallas_call(
        functools.partial(_tc_body, bg=bg, k=k),
        grid=(g // bg,),
        in_specs=[
            pl.BlockSpec((bg, 128), lambda i: (i, 0)),
            pl.BlockSpec((8, k), lambda i: (0, 0)),
        ],
        out_specs=pl.BlockSpec((bg, 128, k), lambda i: (i, 0, 0)),
        out_shape=jax.ShapeDtypeStruct((g, 128, k), jnp.float32),
    )(xx.reshape(g, 128), consts)


def kernel(x, edge_type, means, stds, mul_weight, bias_weight):
    b, n, _ = x.shape
    k = means.shape[0]
    bnn = b * n * n

    xx = _sc_gather_affine(
        x.reshape(bnn),
        edge_type.reshape(bnn),
        mul_weight.reshape(-1),
        bias_weight.reshape(-1),
    )

    a = 1.0 / math.sqrt(2.0 * math.pi)
    log2e = math.log2(math.e)
    inv = 1.0 / (stds + 1e-6)
    consts = jnp.zeros((8, k), jnp.float32)
    consts = consts.at[0].set(means)
    consts = consts.at[1].set(-0.5 * log2e * inv * inv)
    consts = consts.at[2].set(a * inv)

    out = _tc_gaussian(xx, consts, bg=256)
    return out.reshape(b, n, n, k)
